# Initial kernel scaffold; baseline (speedup 1.0000x reference)
import jax
import jax.numpy as jnp
from jax.experimental import pallas as pl


def kernel(x, y):
    raise NotImplementedError("write your pallas kernel here")


# speedup vs baseline: 1.1548x; 1.1548x over previous
import jax
import jax.numpy as jnp
from jax.experimental import pallas as pl
from jax.experimental.pallas import tpu as pltpu

_ROWS = 32768
_COLS = 1024
_BLOCK_ROWS = 1024
_GRID = _ROWS // _BLOCK_ROWS


def _count_kernel(x_ref, y_ref, out_ref, acc_ref):
    i = pl.program_id(0)

    @pl.when(i == 0)
    def _init():
        acc_ref[0] = 0
        acc_ref[1] = 0

    acc_ref[0] += jnp.sum((x_ref[...] > 0).astype(jnp.int32))
    acc_ref[1] += jnp.sum((y_ref[...] > 0).astype(jnp.int32))

    @pl.when(i == _GRID - 1)
    def _finish():
        out_ref[0] = jnp.maximum(acc_ref[0], acc_ref[1])


def kernel(x, y):
    out = pl.pallas_call(
        _count_kernel,
        grid=(_GRID,),
        in_specs=[
            pl.BlockSpec((_BLOCK_ROWS, _COLS), lambda i: (i, 0)),
            pl.BlockSpec((_BLOCK_ROWS, _COLS), lambda i: (i, 0)),
        ],
        out_specs=pl.BlockSpec(memory_space=pltpu.SMEM),
        out_shape=jax.ShapeDtypeStruct((1,), jnp.int32),
        scratch_shapes=[pltpu.SMEM((2,), jnp.int32)],
    )(x, y)
    return out[0]
